# baseline (speedup 1.0000x reference)
import jax
import jax.numpy as jnp
from jax import lax
from jax.experimental import pallas as pl
from jax.experimental.pallas import tpu as pltpu, tpu_sc as plsc

_NC, _NS = 2, 16
_NW = _NC * _NS
_NBUF = 4
_J = 2


def _make_body(seq_len, d, chunks_per_w, gi):
    def body(x_hbm, tok_hbm, pos_hbm, out_hbm,
             idxv, pos_v, gb0, gb1, gb2, gb3, *sems):
        gbufs = (gb0, gb1, gb2, gb3)
        isem = sems[:_NBUF]
        gsem = sems[_NBUF:2 * _NBUF]
        ssem = sems[2 * _NBUF:]
        wid = lax.axis_index("s") * _NC + lax.axis_index("c")
        seq0 = wid * chunks_per_w

        pltpu.sync_copy(pos_hbm, pos_v)

        def fire_idx(g, b):
            pltpu.async_copy(x_hbm.at[seq0 + g], idxv.at[b], isem[b])

        def wait_idx(b):
            pltpu.make_async_copy(x_hbm.at[0], idxv.at[b], isem[b]).wait()

        def fire_gather(b):
            for j in range(_J):
                pltpu.async_copy(
                    tok_hbm.at[idxv.at[b, j]],
                    gbufs[b].at[pl.ds(j * gi, gi)],
                    gsem[b])

        def wait_gather(b):
            pltpu.make_async_copy(
                tok_hbm.at[pl.ds(0, seq_len)], gbufs[b], gsem[b]).wait()

        def add_pos(b):
            gb = gbufs[b]

            @pl.loop(0, seq_len)
            def _(r):
                for c in range(d // 16):
                    sl = pl.ds(c * 16, 16)
                    gb.at[r][sl] = gb[r, sl] + pos_v[r, sl]

        def fire_store(g, b):
            pltpu.async_copy(
                gbufs[b],
                out_hbm.at[pl.ds((seq0 + g) * seq_len, seq_len)],
                ssem[b])

        def wait_store(b):
            pltpu.make_async_copy(
                gbufs[b], out_hbm.at[pl.ds(0, seq_len)], ssem[b]).wait()

        for b in range(_NBUF):
            fire_idx(b, b)

        @pl.loop(0, chunks_per_w, step=_NBUF)
        def _(g):
            for b in range(_NBUF):
                @pl.when(g > 0)
                def _():
                    wait_store(b)
                wait_idx(b)
                fire_gather(b)
            for b in range(_NBUF):
                wait_gather(b)
                add_pos(b)
                fire_store(g + b, b)
                nxt = g + b + _NBUF

                @pl.when(nxt < chunks_per_w)
                def _():
                    fire_idx(nxt, b)

        for b in range(_NBUF):
            wait_store(b)

    return body


def kernel(x, token_emb, pos_emb):
    batch, seq_len = x.shape
    vocab, d = token_emb.shape
    assert batch % (_NW * _NBUF) == 0 and seq_len % _J == 0
    gi = seq_len // _J
    assert gi <= 128 and (gi * d) % 8 == 0
    chunks_per_w = batch // _NW

    x3 = x.astype(jnp.int32).reshape(batch, _J, gi)
    body = _make_body(seq_len, d, chunks_per_w, gi)

    out = pl.kernel(
        body,
        out_type=jax.ShapeDtypeStruct((batch * seq_len, d), jnp.float32),
        mesh=plsc.VectorSubcoreMesh(core_axis_name="c", subcore_axis_name="s"),
        scratch_types=[
            pltpu.VMEM((_NBUF, _J, gi), jnp.int32),
            pltpu.VMEM((seq_len, d), jnp.float32),
        ] + [pltpu.VMEM((seq_len, d), jnp.float32)] * _NBUF
          + [pltpu.SemaphoreType.DMA] * (3 * _NBUF),
        compiler_params=pltpu.CompilerParams(use_tc_tiling_on_sc=False),
    )(x3, token_emb, pos_emb)
    return out.reshape(batch, seq_len, d)


# speedup vs baseline: 1.0021x; 1.0021x over previous
import jax
import jax.numpy as jnp
from jax import lax
from jax.experimental import pallas as pl
from jax.experimental.pallas import tpu as pltpu, tpu_sc as plsc

_NC, _NS = 2, 16
_NW = _NC * _NS
_NBUF = 4
_J = 5


def _make_body(seq_len, d, chunks_per_w, gi):
    def body(x_hbm, tok_hbm, pos_hbm, out_hbm,
             idxv, pos_v, gb0, gb1, gb2, gb3, *sems):
        gbufs = (gb0, gb1, gb2, gb3)
        isem = sems[:_NBUF]
        gsem = sems[_NBUF:2 * _NBUF]
        ssem = sems[2 * _NBUF:]
        wid = lax.axis_index("s") * _NC + lax.axis_index("c")
        seq0 = wid * chunks_per_w

        pltpu.sync_copy(pos_hbm, pos_v)

        def fire_idx(g, b):
            pltpu.async_copy(x_hbm.at[seq0 + g], idxv.at[b], isem[b])

        def wait_idx(b):
            pltpu.make_async_copy(x_hbm.at[0], idxv.at[b], isem[b]).wait()

        def fire_gather(b):
            for j in range(_J):
                pltpu.async_copy(
                    tok_hbm.at[idxv.at[b, pl.ds(j * gi, gi)]],
                    gbufs[b].at[pl.ds(j * gi, gi)],
                    gsem[b])

        def wait_gather(b):
            pltpu.make_async_copy(
                tok_hbm.at[pl.ds(0, seq_len)], gbufs[b], gsem[b]).wait()

        def add_pos(b):
            gb = gbufs[b]

            @pl.loop(0, seq_len)
            def _(r):
                for c in range(d // 16):
                    sl = pl.ds(c * 16, 16)
                    gb.at[r][sl] = gb[r, sl] + pos_v[r, sl]

        def fire_store(g, b):
            pltpu.async_copy(
                gbufs[b],
                out_hbm.at[pl.ds((seq0 + g) * seq_len, seq_len)],
                ssem[b])

        def wait_store(b):
            pltpu.make_async_copy(
                gbufs[b], out_hbm.at[pl.ds(0, seq_len)], ssem[b]).wait()

        for b in range(_NBUF):
            fire_idx(b, b)

        @pl.loop(0, chunks_per_w, step=_NBUF)
        def _(g):
            for b in range(_NBUF):
                @pl.when(g > 0)
                def _():
                    wait_store(b)
                wait_idx(b)
                fire_gather(b)
            for b in range(_NBUF):
                wait_gather(b)
                add_pos(b)
                fire_store(g + b, b)
                nxt = g + b + _NBUF

                @pl.when(nxt < chunks_per_w)
                def _():
                    fire_idx(nxt, b)

        for b in range(_NBUF):
            wait_store(b)

    return body


def kernel(x, token_emb, pos_emb):
    batch, seq_len = x.shape
    vocab, d = token_emb.shape
    assert batch % (_NW * _NBUF) == 0 and seq_len % _J == 0
    gi = seq_len // _J
    assert gi <= 128 and gi % 8 == 0
    chunks_per_w = batch // _NW

    x2 = x.astype(jnp.int32)
    body = _make_body(seq_len, d, chunks_per_w, gi)

    out = pl.kernel(
        body,
        out_type=jax.ShapeDtypeStruct((batch * seq_len, d), jnp.float32),
        mesh=plsc.VectorSubcoreMesh(core_axis_name="c", subcore_axis_name="s"),
        scratch_types=[
            pltpu.VMEM((_NBUF, seq_len), jnp.int32),
            pltpu.VMEM((seq_len, d), jnp.float32),
        ] + [pltpu.VMEM((seq_len, d), jnp.float32)] * _NBUF
          + [pltpu.SemaphoreType.DMA] * (3 * _NBUF),
        compiler_params=pltpu.CompilerParams(use_tc_tiling_on_sc=False),
    )(x2, token_emb, pos_emb)
    return out.reshape(batch, seq_len, d)


# speedup vs baseline: 1.0173x; 1.0152x over previous
import jax
import jax.numpy as jnp
from jax import lax
from jax.experimental import pallas as pl
from jax.experimental.pallas import tpu as pltpu, tpu_sc as plsc

_NC, _NS = 2, 16
_NW = _NC * _NS
_NBUF = 8
_J = 5


def _make_body(seq_len, d, chunks_per_w, gi):
    def body(x_hbm, tok_hbm, pos_hbm, out_hbm,
             idxv, pos_v, gb0, gb1, gb2, gb3, gb4, gb5, gb6, gb7, *sems):
        gbufs = (gb0, gb1, gb2, gb3, gb4, gb5, gb6, gb7)
        isem = sems[:_NBUF]
        gsem = sems[_NBUF:2 * _NBUF]
        ssem = sems[2 * _NBUF:]
        wid = lax.axis_index("s") * _NC + lax.axis_index("c")
        seq0 = wid * chunks_per_w

        pltpu.sync_copy(pos_hbm, pos_v)

        def fire_idx(g, b):
            pltpu.async_copy(x_hbm.at[seq0 + g], idxv.at[b], isem[b])

        def wait_idx(b):
            pltpu.make_async_copy(x_hbm.at[0], idxv.at[b], isem[b]).wait()

        def fire_gather(b):
            for j in range(_J):
                pltpu.async_copy(
                    tok_hbm.at[idxv.at[b, pl.ds(j * gi, gi)]],
                    gbufs[b].at[pl.ds(j * gi, gi)],
                    gsem[b])

        def wait_gather(b):
            pltpu.make_async_copy(
                tok_hbm.at[pl.ds(0, seq_len)], gbufs[b], gsem[b]).wait()

        def add_pos(b):
            gb = gbufs[b]

            @pl.loop(0, seq_len)
            def _(r):
                for c in range(d // 16):
                    sl = pl.ds(c * 16, 16)
                    gb.at[r][sl] = gb[r, sl] + pos_v[r, sl]

        def fire_store(g, b):
            pltpu.async_copy(
                gbufs[b],
                out_hbm.at[pl.ds((seq0 + g) * seq_len, seq_len)],
                ssem[b])

        def wait_store(b):
            pltpu.make_async_copy(
                gbufs[b], out_hbm.at[pl.ds(0, seq_len)], ssem[b]).wait()

        for b in range(_NBUF):
            fire_idx(b, b)

        @pl.loop(0, chunks_per_w, step=_NBUF)
        def _(g):
            for b in range(_NBUF):
                @pl.when(g > 0)
                def _():
                    wait_store(b)
                wait_idx(b)
                fire_gather(b)
            for b in range(_NBUF):
                wait_gather(b)
                add_pos(b)
                fire_store(g + b, b)
                nxt = g + b + _NBUF

                @pl.when(nxt < chunks_per_w)
                def _():
                    fire_idx(nxt, b)

        for b in range(_NBUF):
            wait_store(b)

    return body


def kernel(x, token_emb, pos_emb):
    batch, seq_len = x.shape
    vocab, d = token_emb.shape
    assert batch % (_NW * _NBUF) == 0 and seq_len % _J == 0
    gi = seq_len // _J
    assert gi <= 128 and gi % 8 == 0
    chunks_per_w = batch // _NW

    x2 = x.astype(jnp.int32)
    body = _make_body(seq_len, d, chunks_per_w, gi)

    out = pl.kernel(
        body,
        out_type=jax.ShapeDtypeStruct((batch * seq_len, d), jnp.float32),
        mesh=plsc.VectorSubcoreMesh(core_axis_name="c", subcore_axis_name="s"),
        scratch_types=[
            pltpu.VMEM((_NBUF, seq_len), jnp.int32),
            pltpu.VMEM((seq_len, d), jnp.float32),
        ] + [pltpu.VMEM((seq_len, d), jnp.float32)] * _NBUF
          + [pltpu.SemaphoreType.DMA] * (3 * _NBUF),
        compiler_params=pltpu.CompilerParams(use_tc_tiling_on_sc=False),
    )(x2, token_emb, pos_emb)
    return out.reshape(batch, seq_len, d)


# speedup vs baseline: 1.0259x; 1.0085x over previous
import jax
import jax.numpy as jnp
from jax import lax
from jax.experimental import pallas as pl
from jax.experimental.pallas import tpu as pltpu, tpu_sc as plsc

_NC, _NS = 2, 16
_NW = _NC * _NS
_NBUF = 4
_J = 5


def _make_body(seq_len, d, chunks_per_w, gi):
    rows = 2 * seq_len
    def body(x_hbm, tok_hbm, pos_hbm, out_hbm,
             idxv, pos_v, gb0, gb1, gb2, gb3, *sems):
        gbufs = (gb0, gb1, gb2, gb3)
        isem = sems[:_NBUF]
        gsem = sems[_NBUF:2 * _NBUF]
        ssem = sems[2 * _NBUF:]
        wid = lax.axis_index("s") * _NC + lax.axis_index("c")
        seq0 = wid * chunks_per_w

        pltpu.sync_copy(pos_hbm, pos_v)

        def fire_idx(g, b):
            pltpu.async_copy(x_hbm.at[seq0 + g], idxv.at[b], isem[b])

        def wait_idx(b):
            pltpu.make_async_copy(x_hbm.at[0], idxv.at[b], isem[b]).wait()

        def fire_gather(b):
            for j in range(_J):
                pltpu.async_copy(
                    tok_hbm.at[idxv.at[b, pl.ds(j * gi, gi)]],
                    gbufs[b].at[pl.ds(j * gi, gi)],
                    gsem[b])

        def wait_gather(b):
            pltpu.make_async_copy(
                tok_hbm.at[pl.ds(0, rows)], gbufs[b], gsem[b]).wait()

        def add_pos(b):
            gb = gbufs[b]

            @pl.loop(0, seq_len)
            def _(r):
                for c in range(d // 16):
                    sl = pl.ds(c * 16, 16)
                    pv = pos_v[r, sl]
                    gb.at[r][sl] = gb[r, sl] + pv
                    gb.at[r + seq_len][sl] = gb[r + seq_len, sl] + pv

        def fire_store(g, b):
            pltpu.async_copy(
                gbufs[b],
                out_hbm.at[pl.ds((seq0 + g) * rows, rows)],
                ssem[b])

        def wait_store(b):
            pltpu.make_async_copy(
                gbufs[b], out_hbm.at[pl.ds(0, rows)], ssem[b]).wait()

        for b in range(_NBUF):
            fire_idx(b, b)

        @pl.loop(0, chunks_per_w, step=_NBUF)
        def _(g):
            for b in range(_NBUF):
                @pl.when(g > 0)
                def _():
                    wait_store(b)
                wait_idx(b)
                fire_gather(b)
            for b in range(_NBUF):
                wait_gather(b)
                add_pos(b)
                fire_store(g + b, b)
                nxt = g + b + _NBUF

                @pl.when(nxt < chunks_per_w)
                def _():
                    fire_idx(nxt, b)

        for b in range(_NBUF):
            wait_store(b)

    return body


def kernel(x, token_emb, pos_emb):
    batch, seq_len = x.shape
    vocab, d = token_emb.shape
    assert batch % (2 * _NW * _NBUF) == 0 and (2 * seq_len) % _J == 0
    gi = 2 * seq_len // _J
    assert gi <= 128 and gi % 8 == 0
    chunks_per_w = batch // (2 * _NW)

    x2 = x.astype(jnp.int32).reshape(batch // 2, 2 * seq_len)
    body = _make_body(seq_len, d, chunks_per_w, gi)

    out = pl.kernel(
        body,
        out_type=jax.ShapeDtypeStruct((batch * seq_len, d), jnp.float32),
        mesh=plsc.VectorSubcoreMesh(core_axis_name="c", subcore_axis_name="s"),
        scratch_types=[
            pltpu.VMEM((_NBUF, 2 * seq_len), jnp.int32),
            pltpu.VMEM((seq_len, d), jnp.float32),
        ] + [pltpu.VMEM((2 * seq_len, d), jnp.float32)] * _NBUF
          + [pltpu.SemaphoreType.DMA] * (3 * _NBUF),
        compiler_params=pltpu.CompilerParams(use_tc_tiling_on_sc=False),
    )(x2, token_emb, pos_emb)
    return out.reshape(batch, seq_len, d)
